# baseline (speedup 1.0000x reference)
import functools

import jax
import jax.numpy as jnp
from jax import lax
from jax.experimental import pallas as pl
from jax.experimental.pallas import tpu as pltpu
from jax.experimental.pallas import tpu_sc as plsc

N = 10000
NPAD = 10240
E = 320000
D = 128
DH = D // 2
NC = 2
NS = 16
TILES = NC * NS
B = 128
EPAD = 327680
KD = EPAD // TILES // B
KA = EPAD // NS // B
RPS = NPAD // NS
ZR = 64
NBUF = 5
PF = 3

_f32 = jnp.float32


def _sc_mesh():
    return plsc.VectorSubcoreMesh(core_axis_name="c", subcore_axis_name="s")


def _deg_body(src_hbm, dst_hbm, out_hbm, srcv, dstv, ones, stage, acc_o, acc_i):
    c = lax.axis_index("c")
    s = lax.axis_index("s")
    tile = c * NS + s
    pltpu.sync_copy(src_hbm.at[tile], srcv)
    pltpu.sync_copy(dst_hbm.at[tile], dstv)
    for l in range(B // 16):
        ones[pl.ds(l * 16, 16)] = jnp.ones((16,), _f32)
    for l in range(RPS // 16):
        stage[pl.ds(l * 16, 16)] = jnp.zeros((16,), _f32)
    pltpu.sync_copy(stage, acc_o.at[pl.ds(s * RPS, RPS)])
    pltpu.sync_copy(stage, acc_i.at[pl.ds(s * RPS, RPS)])
    plsc.subcore_barrier()

    @pl.loop(0, KD)
    def _(j):
        pltpu.sync_copy(ones, acc_o.at[srcv.at[j]], add=True)
        pltpu.sync_copy(ones, acc_i.at[dstv.at[j]], add=True)

    plsc.subcore_barrier()
    pltpu.sync_copy(acc_o.at[pl.ds(s * RPS, RPS)],
                    out_hbm.at[c, 0, pl.ds(s * RPS, RPS)])
    pltpu.sync_copy(acc_i.at[pl.ds(s * RPS, RPS)],
                    out_hbm.at[c, 1, pl.ds(s * RPS, RPS)])


def _sc_degrees(src3, dst3):
    kern = pl.kernel(
        _deg_body,
        out_type=jax.ShapeDtypeStruct((NC, 2, NPAD), _f32),
        mesh=_sc_mesh(),
        scratch_types=[
            pltpu.VMEM((KD, B), jnp.int32),
            pltpu.VMEM((KD, B), jnp.int32),
            pltpu.VMEM((B,), _f32),
            pltpu.VMEM((RPS,), _f32),
            pltpu.VMEM_SHARED((NPAD,), _f32),
            pltpu.VMEM_SHARED((NPAD,), _f32),
        ],
    )
    return kern(src3, dst3)


def _agg_body(table_hbm, src_hbm, dst_hbm, out_hbm, srcv, dstv, rows, zstage,
              acc, *sems):
    gsems, ssems = sems[:NBUF], sems[NBUF:]
    c = lax.axis_index("c")
    s = lax.axis_index("s")
    pltpu.sync_copy(src_hbm.at[c, s], srcv)
    pltpu.sync_copy(dst_hbm.at[s], dstv)

    for b in range(PF):
        pltpu.async_copy(table_hbm.at[srcv.at[b]], rows.at[b], gsems[b])

    @pl.loop(0, ZR)
    def _(i):
        for l in range(DH // 16):
            zstage[i, pl.ds(l * 16, 16)] = jnp.zeros((16,), _f32)

    @pl.loop(0, RPS // ZR)
    def _(j):
        pltpu.sync_copy(zstage, acc.at[pl.ds(s * RPS + j * ZR, ZR)])

    plsc.subcore_barrier()

    @pl.loop(0, KA, step=NBUF)
    def _(j):
        for b in range(NBUF):
            chunk = j + b
            pltpu.make_async_copy(table_hbm.at[srcv.at[chunk]],
                                  rows.at[b], gsems[b]).wait()
            pltpu.async_copy(rows.at[b], acc.at[dstv.at[chunk]], ssems[b],
                             add=True)
            bp = (b + PF) % NBUF

            @pl.when(chunk + PF < KA)
            def _():
                @pl.when(chunk + PF >= NBUF)
                def _():
                    pltpu.make_async_copy(
                        rows.at[bp], acc.at[dstv.at[chunk]],
                        ssems[bp]).wait()

                pltpu.async_copy(table_hbm.at[srcv.at[chunk + PF]],
                                 rows.at[bp], gsems[bp])

    for chunk in range(KA - NBUF, KA):
        b = chunk % NBUF
        pltpu.make_async_copy(rows.at[b], acc.at[dstv.at[chunk]],
                              ssems[b]).wait()

    plsc.subcore_barrier()

    for j in range(RPS // ZR):
        pltpu.async_copy(acc.at[pl.ds(s * RPS + j * ZR, ZR)],
                         out_hbm.at[c, pl.ds(s * RPS + j * ZR, ZR)], gsems[0])
    for j in range(RPS // ZR):
        pltpu.make_async_copy(acc.at[pl.ds(s * RPS + j * ZR, ZR)],
                              out_hbm.at[c, pl.ds(s * RPS + j * ZR, ZR)],
                              gsems[0]).wait()


def _sc_aggregate(table2, srcb, dst16):
    kern = pl.kernel(
        _agg_body,
        out_type=jax.ShapeDtypeStruct((NC, NPAD, DH), _f32),
        mesh=_sc_mesh(),
        scratch_types=[
            pltpu.VMEM((KA, B), jnp.int32),
            pltpu.VMEM((KA, B), jnp.int32),
            pltpu.VMEM((NBUF, B, DH), _f32),
            pltpu.VMEM((ZR, DH), _f32),
            pltpu.VMEM_SHARED((NPAD, DH), _f32),
        ] + [pltpu.SemaphoreType.DMA] * (2 * NBUF),
        compiler_params=pltpu.CompilerParams(use_tc_tiling_on_sc=False),
    )
    return kern(table2, srcb, dst16)


R = 1280
NR = NPAD // R


def _inv_sqrt_deg(two_col):
    deg = two_col[:, 0:1] + two_col[:, 1:2]
    return lax.rsqrt(jnp.maximum(deg, 1.0))


def _tc0_body(x_ref, w_ref, do_ref, o_ref):
    t = jnp.dot(x_ref[...], w_ref[0], preferred_element_type=_f32)
    o_ref[...] = t * _inv_sqrt_deg(do_ref[...])


def _split_w(W):
    return jnp.transpose(W.reshape(D, NC, DH), (1, 0, 2))


def _tc_layer0(x_pad, W0, do_p):
    return pl.pallas_call(
        _tc0_body,
        grid=(NR, NC),
        in_specs=[
            pl.BlockSpec((R, D), lambda i, c: (i, 0)),
            pl.BlockSpec((1, D, DH), lambda i, c: (c, 0, 0)),
            pl.BlockSpec((R, NC), lambda i, c: (i, 0)),
        ],
        out_specs=pl.BlockSpec((R, DH), lambda i, c: (c * NR + i, 0)),
        out_shape=jax.ShapeDtypeStruct((NC * NPAD, DH), _f32),
    )(x_pad, W0, do_p)


def _tc1_body(p_ref, di_ref, do_ref, b0_ref, w_ref, o_ref):
    agg = jnp.concatenate([p_ref[0], p_ref[1]], axis=1)
    h = jnp.maximum(agg * _inv_sqrt_deg(di_ref[...]) + b0_ref[...][None, :], 0.0)
    t = jnp.dot(h, w_ref[0], preferred_element_type=_f32)
    o_ref[...] = t * _inv_sqrt_deg(do_ref[...])


def _tc_layer1(p, di_p, do_p, b0, W1):
    return pl.pallas_call(
        _tc1_body,
        grid=(NR, NC),
        in_specs=[
            pl.BlockSpec((NC, R, DH), lambda i, c: (0, i, 0)),
            pl.BlockSpec((R, NC), lambda i, c: (i, 0)),
            pl.BlockSpec((R, NC), lambda i, c: (i, 0)),
            pl.BlockSpec((D,), lambda i, c: (0,)),
            pl.BlockSpec((1, D, DH), lambda i, c: (c, 0, 0)),
        ],
        out_specs=pl.BlockSpec((R, DH), lambda i, c: (c * NR + i, 0)),
        out_shape=jax.ShapeDtypeStruct((NC * NPAD, DH), _f32),
    )(p, di_p, do_p, b0, W1)


RF = 2000


def _tc2_body(p_ref, di_ref, b1_ref, o_ref):
    agg = jnp.concatenate([p_ref[0], p_ref[1]], axis=1)
    o_ref[...] = agg * _inv_sqrt_deg(di_ref[...]) + b1_ref[...][None, :]


def _tc_final(p, di_p, b1):
    return pl.pallas_call(
        _tc2_body,
        grid=(N // RF,),
        in_specs=[
            pl.BlockSpec((NC, RF, DH), lambda i: (0, i, 0)),
            pl.BlockSpec((RF, NC), lambda i: (i, 0)),
            pl.BlockSpec((D,), lambda i: (0,)),
        ],
        out_specs=pl.BlockSpec((RF, D), lambda i: (i, 0)),
        out_shape=jax.ShapeDtypeStruct((N, D), _f32),
    )(p, di_p, b1)


def kernel(x, edge_index, W0, b0, W1, b1):
    pad = jnp.full((EPAD - E,), NPAD - 1, jnp.int32)
    srcf = jnp.concatenate([edge_index[0], pad])
    dstf = jnp.concatenate([edge_index[1], pad])
    src3 = srcf.reshape(TILES, KD, B)
    dst3 = dstf.reshape(TILES, KD, B)
    bias = jnp.array([0, NPAD], jnp.int32)[:, None, None, None]
    srcb = srcf.reshape(NS, KA, B)[None] + bias
    dst16 = dstf.reshape(NS, KA, B)

    degp = _sc_degrees(src3, dst3)
    do_p = jnp.transpose(degp[:, 0, :])
    di_p = jnp.transpose(degp[:, 1, :])

    t0 = _tc_layer0(x, _split_w(W0), do_p)
    p0 = _sc_aggregate(t0, srcb, dst16)
    t1 = _tc_layer1(p0, di_p, do_p, b0, _split_w(W1))
    p1 = _sc_aggregate(t1, srcb, dst16)
    return _tc_final(p1, di_p, b1)


# speedup vs baseline: 1.0517x; 1.0517x over previous
import functools

import jax
import jax.numpy as jnp
from jax import lax
from jax.experimental import pallas as pl
from jax.experimental.pallas import tpu as pltpu
from jax.experimental.pallas import tpu_sc as plsc

N = 10000
NPAD = 10240
E = 320000
D = 128
DH = D // 2
NC = 2
NS = 16
TILES = NC * NS
B = 128
EPAD = 327680
KD = EPAD // TILES // B
KA = EPAD // NS // B
RPS = NPAD // NS
ZR = 64
NBUF = 5
PF = 4

_f32 = jnp.float32


def _sc_mesh():
    return plsc.VectorSubcoreMesh(core_axis_name="c", subcore_axis_name="s")


def _deg_body(src_hbm, dst_hbm, out_hbm, srcv, dstv, ones, stage, acc_o, acc_i):
    c = lax.axis_index("c")
    s = lax.axis_index("s")
    tile = c * NS + s
    pltpu.sync_copy(src_hbm.at[tile], srcv)
    pltpu.sync_copy(dst_hbm.at[tile], dstv)
    for l in range(B // 16):
        ones[pl.ds(l * 16, 16)] = jnp.ones((16,), _f32)
    for l in range(RPS // 16):
        stage[pl.ds(l * 16, 16)] = jnp.zeros((16,), _f32)
    pltpu.sync_copy(stage, acc_o.at[pl.ds(s * RPS, RPS)])
    pltpu.sync_copy(stage, acc_i.at[pl.ds(s * RPS, RPS)])
    plsc.subcore_barrier()

    @pl.loop(0, KD)
    def _(j):
        pltpu.sync_copy(ones, acc_o.at[srcv.at[j]], add=True)
        pltpu.sync_copy(ones, acc_i.at[dstv.at[j]], add=True)

    plsc.subcore_barrier()
    pltpu.sync_copy(acc_o.at[pl.ds(s * RPS, RPS)],
                    out_hbm.at[c, 0, pl.ds(s * RPS, RPS)])
    pltpu.sync_copy(acc_i.at[pl.ds(s * RPS, RPS)],
                    out_hbm.at[c, 1, pl.ds(s * RPS, RPS)])


def _sc_degrees(src3, dst3):
    kern = pl.kernel(
        _deg_body,
        out_type=jax.ShapeDtypeStruct((NC, 2, NPAD), _f32),
        mesh=_sc_mesh(),
        scratch_types=[
            pltpu.VMEM((KD, B), jnp.int32),
            pltpu.VMEM((KD, B), jnp.int32),
            pltpu.VMEM((B,), _f32),
            pltpu.VMEM((RPS,), _f32),
            pltpu.VMEM_SHARED((NPAD,), _f32),
            pltpu.VMEM_SHARED((NPAD,), _f32),
        ],
    )
    return kern(src3, dst3)


def _agg_body(table_hbm, src_hbm, dst_hbm, out_hbm, srcv, dstv, rows, zstage,
              acc, *sems):
    gsems, ssems = sems[:NBUF], sems[NBUF:]
    c = lax.axis_index("c")
    s = lax.axis_index("s")
    pltpu.sync_copy(src_hbm.at[c, s], srcv)
    pltpu.sync_copy(dst_hbm.at[s], dstv)

    for b in range(PF):
        pltpu.async_copy(table_hbm.at[srcv.at[b]], rows.at[b], gsems[b])

    @pl.loop(0, ZR)
    def _(i):
        for l in range(DH // 16):
            zstage[i, pl.ds(l * 16, 16)] = jnp.zeros((16,), _f32)

    @pl.loop(0, RPS // ZR)
    def _(j):
        pltpu.sync_copy(zstage, acc.at[pl.ds(s * RPS + j * ZR, ZR)])

    plsc.subcore_barrier()

    @pl.loop(0, KA, step=NBUF)
    def _(j):
        for b in range(NBUF):
            chunk = j + b
            pltpu.make_async_copy(table_hbm.at[srcv.at[chunk]],
                                  rows.at[b], gsems[b]).wait()
            pltpu.async_copy(rows.at[b], acc.at[dstv.at[chunk]], ssems[b],
                             add=True)
            bp = (b + PF) % NBUF

            @pl.when(chunk + PF < KA)
            def _():
                @pl.when(chunk + PF >= NBUF)
                def _():
                    pltpu.make_async_copy(
                        rows.at[bp], acc.at[dstv.at[chunk]],
                        ssems[bp]).wait()

                pltpu.async_copy(table_hbm.at[srcv.at[chunk + PF]],
                                 rows.at[bp], gsems[bp])

    for chunk in range(KA - NBUF, KA):
        b = chunk % NBUF
        pltpu.make_async_copy(rows.at[b], acc.at[dstv.at[chunk]],
                              ssems[b]).wait()

    plsc.subcore_barrier()

    for j in range(RPS // ZR):
        pltpu.async_copy(acc.at[pl.ds(s * RPS + j * ZR, ZR)],
                         out_hbm.at[c, pl.ds(s * RPS + j * ZR, ZR)], gsems[0])
    for j in range(RPS // ZR):
        pltpu.make_async_copy(acc.at[pl.ds(s * RPS + j * ZR, ZR)],
                              out_hbm.at[c, pl.ds(s * RPS + j * ZR, ZR)],
                              gsems[0]).wait()


def _sc_aggregate(table2, srcb, dst16):
    kern = pl.kernel(
        _agg_body,
        out_type=jax.ShapeDtypeStruct((NC, NPAD, DH), _f32),
        mesh=_sc_mesh(),
        scratch_types=[
            pltpu.VMEM((KA, B), jnp.int32),
            pltpu.VMEM((KA, B), jnp.int32),
            pltpu.VMEM((NBUF, B, DH), _f32),
            pltpu.VMEM((ZR, DH), _f32),
            pltpu.VMEM_SHARED((NPAD, DH), _f32),
        ] + [pltpu.SemaphoreType.DMA] * (2 * NBUF),
        compiler_params=pltpu.CompilerParams(use_tc_tiling_on_sc=False),
    )
    return kern(table2, srcb, dst16)


R = 1280
NR = NPAD // R


def _inv_sqrt_deg(two_col):
    deg = two_col[:, 0:1] + two_col[:, 1:2]
    return lax.rsqrt(jnp.maximum(deg, 1.0))


def _tc0_body(x_ref, w_ref, do_ref, o_ref):
    t = jnp.dot(x_ref[...], w_ref[0], preferred_element_type=_f32)
    o_ref[...] = t * _inv_sqrt_deg(do_ref[...])


def _split_w(W):
    return jnp.transpose(W.reshape(D, NC, DH), (1, 0, 2))


def _tc_layer0(x_pad, W0, do_p):
    return pl.pallas_call(
        _tc0_body,
        grid=(NR, NC),
        in_specs=[
            pl.BlockSpec((R, D), lambda i, c: (i, 0)),
            pl.BlockSpec((1, D, DH), lambda i, c: (c, 0, 0)),
            pl.BlockSpec((R, NC), lambda i, c: (i, 0)),
        ],
        out_specs=pl.BlockSpec((R, DH), lambda i, c: (c * NR + i, 0)),
        out_shape=jax.ShapeDtypeStruct((NC * NPAD, DH), _f32),
    )(x_pad, W0, do_p)


def _tc1_body(p_ref, di_ref, do_ref, b0_ref, w_ref, o_ref):
    agg = jnp.concatenate([p_ref[0], p_ref[1]], axis=1)
    h = jnp.maximum(agg * _inv_sqrt_deg(di_ref[...]) + b0_ref[...][None, :], 0.0)
    t = jnp.dot(h, w_ref[0], preferred_element_type=_f32)
    o_ref[...] = t * _inv_sqrt_deg(do_ref[...])


def _tc_layer1(p, di_p, do_p, b0, W1):
    return pl.pallas_call(
        _tc1_body,
        grid=(NR, NC),
        in_specs=[
            pl.BlockSpec((NC, R, DH), lambda i, c: (0, i, 0)),
            pl.BlockSpec((R, NC), lambda i, c: (i, 0)),
            pl.BlockSpec((R, NC), lambda i, c: (i, 0)),
            pl.BlockSpec((D,), lambda i, c: (0,)),
            pl.BlockSpec((1, D, DH), lambda i, c: (c, 0, 0)),
        ],
        out_specs=pl.BlockSpec((R, DH), lambda i, c: (c * NR + i, 0)),
        out_shape=jax.ShapeDtypeStruct((NC * NPAD, DH), _f32),
    )(p, di_p, do_p, b0, W1)


def _tc2_body(p_ref, di_ref, b1_ref, o_ref):
    agg = jnp.concatenate([p_ref[0], p_ref[1]], axis=1)
    o_ref[...] = agg * _inv_sqrt_deg(di_ref[...]) + b1_ref[...][None, :]


def _tc_final(p, di_p, b1):
    return pl.pallas_call(
        _tc2_body,
        grid=(NR,),
        in_specs=[
            pl.BlockSpec((NC, R, DH), lambda i: (0, i, 0)),
            pl.BlockSpec((R, NC), lambda i: (i, 0)),
            pl.BlockSpec((D,), lambda i: (0,)),
        ],
        out_specs=pl.BlockSpec((R, D), lambda i: (i, 0)),
        out_shape=jax.ShapeDtypeStruct((NPAD, D), _f32),
    )(p, di_p, b1)


def kernel(x, edge_index, W0, b0, W1, b1):
    pad = jnp.full((EPAD - E,), NPAD - 1, jnp.int32)
    srcf = jnp.concatenate([edge_index[0], pad])
    dstf = jnp.concatenate([edge_index[1], pad])
    src3 = srcf.reshape(TILES, KD, B)
    dst3 = dstf.reshape(TILES, KD, B)
    bias = jnp.array([0, NPAD], jnp.int32)[:, None, None, None]
    srcb = srcf.reshape(NS, KA, B)[None] + bias
    dst16 = dstf.reshape(NS, KA, B)
    x_pad = jnp.pad(x, ((0, NPAD - N), (0, 0)))

    degp = _sc_degrees(src3, dst3)
    do_p = jnp.transpose(degp[:, 0, :])
    di_p = jnp.transpose(degp[:, 1, :])

    t0 = _tc_layer0(x_pad, _split_w(W0), do_p)
    p0 = _sc_aggregate(t0, srcb, dst16)
    t1 = _tc_layer1(p0, di_p, do_p, b0, _split_w(W1))
    p1 = _sc_aggregate(t1, srcb, dst16)
    out = _tc_final(p1, di_p, b1)
    return out[:N]


# speedup vs baseline: 1.0561x; 1.0042x over previous
import functools

import jax
import jax.numpy as jnp
from jax import lax
from jax.experimental import pallas as pl
from jax.experimental.pallas import tpu as pltpu
from jax.experimental.pallas import tpu_sc as plsc

N = 10000
NPAD = 10240
E = 320000
D = 128
DH = D // 2
NC = 2
NS = 16
TILES = NC * NS
B = 128
EPAD = 327680
KD = EPAD // TILES // B
KA = EPAD // NS // B
RPS = NPAD // NS
ZR = 64
NBUF = 5
PF = 4

_f32 = jnp.float32


def _sc_mesh():
    return plsc.VectorSubcoreMesh(core_axis_name="c", subcore_axis_name="s")


def _deg_body(src_hbm, dst_hbm, out_hbm, srcv, dstv, ones, stage, acc_o, acc_i,
              osem, isem):
    c = lax.axis_index("c")
    s = lax.axis_index("s")
    tile = c * NS + s
    pltpu.sync_copy(src_hbm.at[tile], srcv)
    pltpu.sync_copy(dst_hbm.at[tile], dstv)
    for l in range(B // 16):
        ones[pl.ds(l * 16, 16)] = jnp.ones((16,), _f32)
    for l in range(RPS // 16):
        stage[pl.ds(l * 16, 16)] = jnp.zeros((16,), _f32)
    pltpu.sync_copy(stage, acc_o.at[pl.ds(s * RPS, RPS)])
    pltpu.sync_copy(stage, acc_i.at[pl.ds(s * RPS, RPS)])
    plsc.subcore_barrier()

    @pl.loop(0, KD, step=4)
    def _(j):
        for b in range(4):
            pltpu.async_copy(ones, acc_o.at[srcv.at[j + b]], osem, add=True)
            pltpu.async_copy(ones, acc_i.at[dstv.at[j + b]], isem, add=True)
        for b in range(4):
            pltpu.make_async_copy(ones, acc_o.at[srcv.at[j + b]], osem).wait()
            pltpu.make_async_copy(ones, acc_i.at[dstv.at[j + b]], isem).wait()

    plsc.subcore_barrier()
    pltpu.sync_copy(acc_o.at[pl.ds(s * RPS, RPS)],
                    out_hbm.at[c, 0, pl.ds(s * RPS, RPS)])
    pltpu.sync_copy(acc_i.at[pl.ds(s * RPS, RPS)],
                    out_hbm.at[c, 1, pl.ds(s * RPS, RPS)])


def _sc_degrees(src3, dst3):
    kern = pl.kernel(
        _deg_body,
        out_type=jax.ShapeDtypeStruct((NC, 2, NPAD), _f32),
        mesh=_sc_mesh(),
        scratch_types=[
            pltpu.VMEM((KD, B), jnp.int32),
            pltpu.VMEM((KD, B), jnp.int32),
            pltpu.VMEM((B,), _f32),
            pltpu.VMEM((RPS,), _f32),
            pltpu.VMEM_SHARED((NPAD,), _f32),
            pltpu.VMEM_SHARED((NPAD,), _f32),
            pltpu.SemaphoreType.DMA,
            pltpu.SemaphoreType.DMA,
        ],
    )
    return kern(src3, dst3)


def _agg_body(table_hbm, src_hbm, dst_hbm, out_hbm, srcv, dstv, rows, zstage,
              acc, *sems):
    gsems, ssems = sems[:NBUF], sems[NBUF:]
    c = lax.axis_index("c")
    s = lax.axis_index("s")
    pltpu.sync_copy(src_hbm.at[c, s], srcv)
    pltpu.sync_copy(dst_hbm.at[s], dstv)

    for b in range(PF):
        pltpu.async_copy(table_hbm.at[srcv.at[b]], rows.at[b], gsems[b])

    @pl.loop(0, ZR)
    def _(i):
        for l in range(DH // 16):
            zstage[i, pl.ds(l * 16, 16)] = jnp.zeros((16,), _f32)

    @pl.loop(0, RPS // ZR)
    def _(j):
        pltpu.sync_copy(zstage, acc.at[pl.ds(s * RPS + j * ZR, ZR)])

    plsc.subcore_barrier()

    @pl.loop(0, KA, step=NBUF)
    def _(j):
        for b in range(NBUF):
            chunk = j + b
            pltpu.make_async_copy(table_hbm.at[srcv.at[chunk]],
                                  rows.at[b], gsems[b]).wait()
            pltpu.async_copy(rows.at[b], acc.at[dstv.at[chunk]], ssems[b],
                             add=True)
            bp = (b + PF) % NBUF

            @pl.when(chunk + PF < KA)
            def _():
                @pl.when(chunk + PF >= NBUF)
                def _():
                    pltpu.make_async_copy(
                        rows.at[bp], acc.at[dstv.at[chunk]],
                        ssems[bp]).wait()

                pltpu.async_copy(table_hbm.at[srcv.at[chunk + PF]],
                                 rows.at[bp], gsems[bp])

    for chunk in range(KA - NBUF, KA):
        b = chunk % NBUF
        pltpu.make_async_copy(rows.at[b], acc.at[dstv.at[chunk]],
                              ssems[b]).wait()

    plsc.subcore_barrier()

    for j in range(RPS // ZR):
        pltpu.async_copy(acc.at[pl.ds(s * RPS + j * ZR, ZR)],
                         out_hbm.at[c, pl.ds(s * RPS + j * ZR, ZR)], gsems[0])
    for j in range(RPS // ZR):
        pltpu.make_async_copy(acc.at[pl.ds(s * RPS + j * ZR, ZR)],
                              out_hbm.at[c, pl.ds(s * RPS + j * ZR, ZR)],
                              gsems[0]).wait()


def _sc_aggregate(table2, srcb, dst16):
    kern = pl.kernel(
        _agg_body,
        out_type=jax.ShapeDtypeStruct((NC, NPAD, DH), _f32),
        mesh=_sc_mesh(),
        scratch_types=[
            pltpu.VMEM((KA, B), jnp.int32),
            pltpu.VMEM((KA, B), jnp.int32),
            pltpu.VMEM((NBUF, B, DH), _f32),
            pltpu.VMEM((ZR, DH), _f32),
            pltpu.VMEM_SHARED((NPAD, DH), _f32),
        ] + [pltpu.SemaphoreType.DMA] * (2 * NBUF),
        compiler_params=pltpu.CompilerParams(use_tc_tiling_on_sc=False),
    )
    return kern(table2, srcb, dst16)


R = 1280
NR = NPAD // R


def _inv_sqrt_deg(two_col):
    deg = two_col[:, 0:1] + two_col[:, 1:2]
    return lax.rsqrt(jnp.maximum(deg, 1.0))


def _tc0_body(x_ref, w_ref, do_ref, o_ref):
    t = jnp.dot(x_ref[...], w_ref[0], preferred_element_type=_f32)
    o_ref[...] = t * _inv_sqrt_deg(do_ref[...])


def _split_w(W):
    return jnp.transpose(W.reshape(D, NC, DH), (1, 0, 2))


def _tc_layer0(x_pad, W0, do_p):
    return pl.pallas_call(
        _tc0_body,
        grid=(NR, NC),
        in_specs=[
            pl.BlockSpec((R, D), lambda i, c: (i, 0)),
            pl.BlockSpec((1, D, DH), lambda i, c: (c, 0, 0)),
            pl.BlockSpec((R, NC), lambda i, c: (i, 0)),
        ],
        out_specs=pl.BlockSpec((R, DH), lambda i, c: (c * NR + i, 0)),
        out_shape=jax.ShapeDtypeStruct((NC * NPAD, DH), _f32),
    )(x_pad, W0, do_p)


def _tc1_body(p_ref, di_ref, do_ref, b0_ref, w_ref, o_ref):
    agg = jnp.concatenate([p_ref[0], p_ref[1]], axis=1)
    h = jnp.maximum(agg * _inv_sqrt_deg(di_ref[...]) + b0_ref[...][None, :], 0.0)
    t = jnp.dot(h, w_ref[0], preferred_element_type=_f32)
    o_ref[...] = t * _inv_sqrt_deg(do_ref[...])


def _tc_layer1(p, di_p, do_p, b0, W1):
    return pl.pallas_call(
        _tc1_body,
        grid=(NR, NC),
        in_specs=[
            pl.BlockSpec((NC, R, DH), lambda i, c: (0, i, 0)),
            pl.BlockSpec((R, NC), lambda i, c: (i, 0)),
            pl.BlockSpec((R, NC), lambda i, c: (i, 0)),
            pl.BlockSpec((D,), lambda i, c: (0,)),
            pl.BlockSpec((1, D, DH), lambda i, c: (c, 0, 0)),
        ],
        out_specs=pl.BlockSpec((R, DH), lambda i, c: (c * NR + i, 0)),
        out_shape=jax.ShapeDtypeStruct((NC * NPAD, DH), _f32),
    )(p, di_p, do_p, b0, W1)


def _tc2_body(p_ref, di_ref, b1_ref, o_ref):
    agg = jnp.concatenate([p_ref[0], p_ref[1]], axis=1)
    o_ref[...] = agg * _inv_sqrt_deg(di_ref[...]) + b1_ref[...][None, :]


def _tc_final(p, di_p, b1):
    return pl.pallas_call(
        _tc2_body,
        grid=(NR,),
        in_specs=[
            pl.BlockSpec((NC, R, DH), lambda i: (0, i, 0)),
            pl.BlockSpec((R, NC), lambda i: (i, 0)),
            pl.BlockSpec((D,), lambda i: (0,)),
        ],
        out_specs=pl.BlockSpec((R, D), lambda i: (i, 0)),
        out_shape=jax.ShapeDtypeStruct((NPAD, D), _f32),
    )(p, di_p, b1)


def kernel(x, edge_index, W0, b0, W1, b1):
    pad = jnp.full((EPAD - E,), NPAD - 1, jnp.int32)
    srcf = jnp.concatenate([edge_index[0], pad])
    dstf = jnp.concatenate([edge_index[1], pad])
    src3 = srcf.reshape(TILES, KD, B)
    dst3 = dstf.reshape(TILES, KD, B)
    bias = jnp.array([0, NPAD], jnp.int32)[:, None, None, None]
    srcb = srcf.reshape(NS, KA, B)[None] + bias
    dst16 = dstf.reshape(NS, KA, B)
    x_pad = jnp.pad(x, ((0, NPAD - N), (0, 0)))

    degp = _sc_degrees(src3, dst3)
    do_p = jnp.transpose(degp[:, 0, :])
    di_p = jnp.transpose(degp[:, 1, :])

    t0 = _tc_layer0(x_pad, _split_w(W0), do_p)
    p0 = _sc_aggregate(t0, srcb, dst16)
    t1 = _tc_layer1(p0, di_p, do_p, b0, _split_w(W1))
    p1 = _sc_aggregate(t1, srcb, dst16)
    out = _tc_final(p1, di_p, b1)
    return out[:N]
